# baseline (speedup 1.0000x reference)
import functools

import jax
import jax.numpy as jnp
from jax.experimental import pallas as pl
from jax.experimental.pallas import tpu as pltpu

_K = 1024
_D = 32
_TILE = 4096
_SUB = 128


def _vq_kernel(x_ref, w_ref, out_ref, loss_ref,
               counts_ref, sq_ref):
    i = pl.program_id(0)
    nsteps = pl.num_programs(0)

    @pl.when(i == 0)
    def _init():
        counts_ref[...] = jnp.zeros_like(counts_ref)
        sq_ref[...] = jnp.zeros_like(sq_ref)

    w = w_ref[...]
    iota = jax.lax.broadcasted_iota(jnp.int32, (1, _K), 1).astype(jnp.float32)
    b_col = jnp.sum(w * w, axis=1, keepdims=True)
    b = jnp.swapaxes(b_col, 0, 1)

    nsub = _TILE // _SUB

    spb = x_ref.shape[1] // _SUB

    def _xs(k):
        return x_ref[k // spb, pl.ds((k % spb) * _SUB, _SUB), :]

    def _mm(k):
        return jax.lax.dot_general(_xs(k) * -2.0, w,
                                   dimension_numbers=(((1,), (1,)), ((), ())))

    csums = []
    sqs = []
    c2_next = _mm(0)
    for k in range(nsub):
        c2 = c2_next
        if k + 1 < nsub:
            c2_next = _mm(k + 1)
        x_s = _xs(k)
        a_s = jnp.sum(x_s * x_s, axis=1, keepdims=True)
        d = (a_s + b) + c2
        m = jnp.min(d, axis=1, keepdims=True)
        sel = jnp.where(d == m, iota, float(_K))
        amin = jnp.min(sel, axis=1, keepdims=True)
        onehot = (iota == amin).astype(jnp.float32)
        q = jax.lax.dot_general(onehot, w,
                                dimension_numbers=(((1,), (0,)), ((), ())))
        diff = q - x_s
        out_ref[k // spb, pl.ds((k % spb) * _SUB, _SUB), :] = x_s + diff
        ones_row = jnp.ones((1, _SUB), jnp.float32)
        csums.append(jax.lax.dot_general(
            ones_row, onehot, dimension_numbers=(((1,), (0,)), ((), ()))))
        sqs.append(jnp.sum(diff * diff, axis=0, keepdims=True))

    counts_ref[...] = counts_ref[...] + sum(csums)
    sq_ref[...] = sq_ref[...] + sum(sqs)

    @pl.when(i == nsteps - 1)
    def _finalize():
        n_total = nsteps * _TILE
        p = counts_ref[...] * (1.0 / n_total)
        mse = jnp.sum(sq_ref[...]) / (n_total * _D)
        loss = mse + 0.25 * mse
        entropy = -jnp.sum(p * jnp.log(p + 1e-10))
        div = jnp.sum((p - 1.0 / _K) ** 2)
        kl = jnp.sum(p * jnp.log(p * float(_K) + 1e-10))
        loss_ref[0, 0] = ((loss - entropy) + div) + kl


@functools.partial(jax.jit)
def kernel(x, W):
    batch, tokens, _ = x.shape
    slabs = _TILE // tokens
    out_q, out_loss = pl.pallas_call(
        _vq_kernel,
        grid=(batch // slabs,),
        in_specs=[
            pl.BlockSpec((slabs, tokens, _D), lambda i: (i, 0, 0)),
            pl.BlockSpec((_K, _D), lambda i: (0, 0)),
        ],
        out_specs=[
            pl.BlockSpec((slabs, tokens, _D), lambda i: (i, 0, 0)),
            pl.BlockSpec(memory_space=pltpu.SMEM),
        ],
        out_shape=[
            jax.ShapeDtypeStruct(x.shape, jnp.float32),
            jax.ShapeDtypeStruct((1, 1), jnp.float32),
        ],
        scratch_shapes=[
            pltpu.VMEM((1, _K), jnp.float32),
            pltpu.VMEM((1, _D), jnp.float32),
        ],
    )(x, W)
    return out_q, out_loss.reshape(())


# speedup vs baseline: 1.3586x; 1.3586x over previous
import functools

import jax
import jax.numpy as jnp
from jax.experimental import pallas as pl
from jax.experimental.pallas import tpu as pltpu

_K = 1024
_D = 32
_TILE = 4096
_SUB = 128


def _vq_kernel(x_ref, w_ref, out_ref, loss_ref, counts_ref, sq_ref):
    i = pl.program_id(0)
    nsteps = pl.num_programs(0)

    @pl.when(i == 0)
    def _init():
        counts_ref[...] = jnp.zeros_like(counts_ref)
        sq_ref[...] = jnp.zeros_like(sq_ref)

    wt = w_ref[...]
    b_row = jnp.sum(wt * wt, axis=0, keepdims=True)
    b_col = jnp.swapaxes(b_row, 0, 1)
    iota_col = jax.lax.broadcasted_iota(
        jnp.int32, (_K, 1), 0).astype(jnp.float32)

    spb = x_ref.shape[2] // _SUB

    def _xs(k):
        return x_ref[k // spb, :, pl.ds((k % spb) * _SUB, _SUB)]

    def _mm(k):
        return jax.lax.dot_general(wt, _xs(k) * -2.0,
                                   dimension_numbers=(((0,), (0,)), ((), ())))

    csums = []
    sqs = []
    c2_next = _mm(0)
    nsub = (x_ref.shape[0] * x_ref.shape[2]) // _SUB
    for k in range(nsub):
        c2 = c2_next
        if k + 1 < nsub:
            c2_next = _mm(k + 1)
        x_s = _xs(k)
        a_row = jnp.sum(x_s * x_s, axis=0, keepdims=True)
        d = (a_row + b_col) + c2
        m = jnp.min(d, axis=0, keepdims=True)
        sel = jnp.where(d == m, iota_col, float(_K))
        amin = jnp.min(sel, axis=0, keepdims=True)
        onehot = (iota_col == amin).astype(jnp.float32)
        q = jax.lax.dot_general(wt, onehot,
                                dimension_numbers=(((1,), (0,)), ((), ())))
        diff = q - x_s
        out_ref[k // spb, :, pl.ds((k % spb) * _SUB, _SUB)] = x_s + diff
        ones_col = jnp.ones((_SUB, 1), jnp.float32)
        csums.append(jax.lax.dot_general(
            onehot, ones_col, dimension_numbers=(((1,), (0,)), ((), ()))))
        sqs.append(jnp.sum(diff * diff, axis=1, keepdims=True))

    counts_ref[...] = counts_ref[...] + sum(csums)
    sq_ref[...] = sq_ref[...] + sum(sqs)

    @pl.when(i == nsteps - 1)
    def _finalize():
        n_total = nsteps * _TILE
        p = counts_ref[...] * (1.0 / n_total)
        mse = jnp.sum(sq_ref[...]) / (n_total * _D)
        loss = mse + 0.25 * mse
        entropy = -jnp.sum(p * jnp.log(p + 1e-10))
        div = jnp.sum((p - 1.0 / _K) ** 2)
        kl = jnp.sum(p * jnp.log(p * float(_K) + 1e-10))
        loss_ref[0, 0] = ((loss - entropy) + div) + kl


@functools.partial(jax.jit)
def kernel(x, W):
    batch, tokens, _ = x.shape
    xt = jnp.transpose(x, (0, 2, 1))
    wt = jnp.transpose(W, (1, 0))
    slabs = _TILE // tokens
    out_t, out_loss = pl.pallas_call(
        _vq_kernel,
        grid=(batch // slabs,),
        in_specs=[
            pl.BlockSpec((slabs, _D, tokens), lambda i: (i, 0, 0)),
            pl.BlockSpec((_D, _K), lambda i: (0, 0)),
        ],
        out_specs=[
            pl.BlockSpec((slabs, _D, tokens), lambda i: (i, 0, 0)),
            pl.BlockSpec(memory_space=pltpu.SMEM),
        ],
        out_shape=[
            jax.ShapeDtypeStruct((batch, _D, tokens), jnp.float32),
            jax.ShapeDtypeStruct((1, 1), jnp.float32),
        ],
        scratch_shapes=[
            pltpu.VMEM((_K, 1), jnp.float32),
            pltpu.VMEM((_D, 1), jnp.float32),
        ],
    )(xt, wt)
    return jnp.transpose(out_t, (0, 2, 1)), out_loss.reshape(())


# speedup vs baseline: 1.4161x; 1.0424x over previous
import functools

import jax
import jax.numpy as jnp
from jax.experimental import pallas as pl
from jax.experimental.pallas import tpu as pltpu

_K = 1024
_D = 32
_TILE = 8192
_SUB = 128


def _vq_kernel(x_ref, w_ref, out_ref, loss_ref, counts_ref, sq_ref):
    i = pl.program_id(0)
    nsteps = pl.num_programs(0)

    @pl.when(i == 0)
    def _init():
        counts_ref[...] = jnp.zeros_like(counts_ref)
        sq_ref[...] = jnp.zeros_like(sq_ref)

    wt = w_ref[...]
    b_row = jnp.sum(wt * wt, axis=0, keepdims=True)
    b_col = jnp.swapaxes(b_row, 0, 1)
    iota_col = jax.lax.broadcasted_iota(
        jnp.int32, (_K, 1), 0).astype(jnp.float32)

    spb = x_ref.shape[2] // _SUB

    def _xs(k):
        return x_ref[k // spb, :, pl.ds((k % spb) * _SUB, _SUB)]

    def _mm(k):
        return jax.lax.dot_general(wt, _xs(k) * -2.0,
                                   dimension_numbers=(((0,), (0,)), ((), ())))

    csums = []
    sqs = []
    c2_next = _mm(0)
    nsub = (x_ref.shape[0] * x_ref.shape[2]) // _SUB
    for k in range(nsub):
        c2 = c2_next
        if k + 1 < nsub:
            c2_next = _mm(k + 1)
        x_s = _xs(k)
        a_row = jnp.sum(x_s * x_s, axis=0, keepdims=True)
        d = (a_row + b_col) + c2
        m = jnp.min(d, axis=0, keepdims=True)
        sel = jnp.where(d == m, iota_col, float(_K))
        amin = jnp.min(sel, axis=0, keepdims=True)
        onehot = (iota_col == amin).astype(jnp.float32)
        q = jax.lax.dot_general(wt, onehot,
                                dimension_numbers=(((1,), (0,)), ((), ())))
        diff = q - x_s
        out_ref[k // spb, :, pl.ds((k % spb) * _SUB, _SUB)] = x_s + diff
        ones_col = jnp.ones((_SUB, 1), jnp.float32)
        csums.append(jax.lax.dot_general(
            onehot, ones_col, dimension_numbers=(((1,), (0,)), ((), ()))))
        sqs.append(jnp.sum(diff * diff, axis=1, keepdims=True))

    counts_ref[...] = counts_ref[...] + sum(csums)
    sq_ref[...] = sq_ref[...] + sum(sqs)

    @pl.when(i == nsteps - 1)
    def _finalize():
        n_total = nsteps * _TILE
        p = counts_ref[...] * (1.0 / n_total)
        mse = jnp.sum(sq_ref[...]) / (n_total * _D)
        loss = mse + 0.25 * mse
        entropy = -jnp.sum(p * jnp.log(p + 1e-10))
        div = jnp.sum((p - 1.0 / _K) ** 2)
        kl = jnp.sum(p * jnp.log(p * float(_K) + 1e-10))
        loss_ref[0, 0] = ((loss - entropy) + div) + kl


@functools.partial(jax.jit)
def kernel(x, W):
    batch, tokens, _ = x.shape
    xt = jnp.transpose(x, (0, 2, 1))
    wt = jnp.transpose(W, (1, 0))
    slabs = _TILE // tokens
    out_t, out_loss = pl.pallas_call(
        _vq_kernel,
        grid=(batch // slabs,),
        in_specs=[
            pl.BlockSpec((slabs, _D, tokens), lambda i: (i, 0, 0)),
            pl.BlockSpec((_D, _K), lambda i: (0, 0)),
        ],
        out_specs=[
            pl.BlockSpec((slabs, _D, tokens), lambda i: (i, 0, 0)),
            pl.BlockSpec(memory_space=pltpu.SMEM),
        ],
        out_shape=[
            jax.ShapeDtypeStruct((batch, _D, tokens), jnp.float32),
            jax.ShapeDtypeStruct((1, 1), jnp.float32),
        ],
        scratch_shapes=[
            pltpu.VMEM((_K, 1), jnp.float32),
            pltpu.VMEM((_D, 1), jnp.float32),
        ],
    )(xt, wt)
    return jnp.transpose(out_t, (0, 2, 1)), out_loss.reshape(())
